# baseline (speedup 1.0000x reference)
import functools

import jax
import jax.numpy as jnp
from jax import lax
from jax.experimental import pallas as pl
from jax.experimental.pallas import tpu as pltpu
from jax.experimental.pallas import tpu_sc as plsc

HIDDEN = 1024
NUM_TOKENS = 4 * 8192

_info = plsc.get_sparse_core_info()
NC, NS = _info.num_cores, _info.num_subcores
NW = NC * NS
TOK_PER_W = NUM_TOKENS // NW
CHUNK = 64
NSTEPS = TOK_PER_W // CHUNK


def _body(table_hbm, idx_hbm, out_hbm, idx_v, rows_v, sem):
    wid = lax.axis_index("s") * NC + lax.axis_index("c")
    base = wid * TOK_PER_W

    def step(i, _):
        off = base + i * CHUNK
        pltpu.sync_copy(idx_hbm.at[pl.ds(off, CHUNK)], idx_v)
        pltpu.async_copy(table_hbm.at[idx_v], rows_v, sem).wait()
        pltpu.sync_copy(rows_v, out_hbm.at[pl.ds(off, CHUNK)])
        return ()

    lax.fori_loop(0, NSTEPS, step, (), unroll=False)


@jax.jit
def _lookup(table, idx_flat):
    mesh = plsc.VectorSubcoreMesh(core_axis_name="c", subcore_axis_name="s")
    return pl.kernel(
        _body,
        out_type=jax.ShapeDtypeStruct((NUM_TOKENS, HIDDEN), jnp.float32),
        mesh=mesh,
        scratch_types=[
            pltpu.VMEM((CHUNK,), jnp.int32),
            pltpu.VMEM((CHUNK, HIDDEN), jnp.float32),
            pltpu.SemaphoreType.DMA,
        ],
    )(table, idx_flat)


def kernel(indices, weight):
    idx_flat = indices.reshape(-1).astype(jnp.int32)
    out = _lookup(weight, idx_flat)
    return out.reshape(*indices.shape, HIDDEN)


# speedup vs baseline: 10.1491x; 10.1491x over previous
import jax
import jax.numpy as jnp
from jax import lax
from jax.experimental import pallas as pl
from jax.experimental.pallas import tpu as pltpu
from jax.experimental.pallas import tpu_sc as plsc

HIDDEN = 1024
NUM_TOKENS = 4 * 8192

_info = plsc.get_sparse_core_info()
NC, NS = _info.num_cores, _info.num_subcores
NW = NC * NS
TOK_PER_W = NUM_TOKENS // NW
K = 16
NGROUPS = TOK_PER_W // K


def _body(table_hbm, idx_hbm, out_hbm, table_v, idx_v, sem):
    wid = lax.axis_index("s") * NC + lax.axis_index("c")
    base = wid * TOK_PER_W

    pltpu.sync_copy(table_hbm, table_v)
    pltpu.sync_copy(idx_hbm.at[pl.ds(base, TOK_PER_W)], idx_v)

    def fire(g):
        vec = idx_v[pl.ds(g * K, K)]
        for j in range(K):
            row = vec[j]
            pltpu.make_async_copy(
                table_v.at[row], out_hbm.at[base + g * K + j], sem
            ).start()

    def drain():
        for _ in range(K):
            pltpu.make_async_copy(
                table_v.at[0], out_hbm.at[base], sem
            ).wait()

    def step(g, _):
        fire(g)

        @pl.when(g > 0)
        def _():
            drain()

        return ()

    lax.fori_loop(0, NGROUPS, step, (), unroll=False)
    drain()


@jax.jit
def _lookup(table, idx_flat):
    mesh = plsc.VectorSubcoreMesh(core_axis_name="c", subcore_axis_name="s")
    return pl.kernel(
        _body,
        out_type=jax.ShapeDtypeStruct((NUM_TOKENS, HIDDEN), jnp.float32),
        mesh=mesh,
        scratch_types=[
            pltpu.VMEM((3, HIDDEN), jnp.float32),
            pltpu.VMEM((TOK_PER_W,), jnp.int32),
            pltpu.SemaphoreType.DMA,
        ],
    )(table, idx_flat)


def kernel(indices, weight):
    idx_flat = indices.reshape(-1).astype(jnp.int32)
    out = _lookup(weight, idx_flat)
    return out.reshape(*indices.shape, HIDDEN)
